# baseline (speedup 1.0000x reference)
import dataclasses
import functools

import jax
import jax.numpy as jnp
from jax import lax
from jax.experimental import pallas as pl
from jax.experimental.pallas import tpu as pltpu
from jax.experimental.pallas import tpu_sc as plsc

B = 4096
D = 64
NT = 1000000
NP = NT // 2
NC, NS = 2, 16
NW = NC * NS
BU = B // NW
BI = 2 * B // NW
BM = 256
TC_C = 4096
GW = 128
L = 16


def _t_body(t_ref, eye_ref, out_ref):
    a = t_ref[...]
    t = jax.lax.dot_general(
        a, eye_ref[...], (((0,), (0,)), ((), ())),
        preferred_element_type=jnp.float32,
    )
    t16 = t.astype(jnp.bfloat16)
    out_ref[...] = pltpu.bitcast(t16, jnp.int32)


def _stage_table(table_t, eye):
    n_steps = (NT + TC_C - 1) // TC_C
    return pl.pallas_call(
        _t_body,
        grid=(n_steps,),
        compiler_params=pltpu.CompilerParams(
            dimension_semantics=("parallel",)
        ),
        in_specs=[
            pl.BlockSpec((D, TC_C), lambda i: (0, i)),
            pl.BlockSpec((D, 128), lambda i: (0, 0)),
        ],
        out_specs=pl.BlockSpec((TC_C // 2, 128), lambda i: (i, 0)),
        out_shape=jax.ShapeDtypeStruct((NP, 128), jnp.int32),
    )(table_t, eye)


def _sc_gather(st_user, st_item, user_id, item_ids2):
    mesh = plsc.VectorSubcoreMesh(core_axis_name="c", subcore_axis_name="s")
    cp = pltpu.CompilerParams()
    if "use_tc_tiling_on_sc" in pltpu.CompilerParams.__dataclass_fields__:
        cp = dataclasses.replace(cp, use_tc_tiling_on_sc=True)

    @functools.partial(
        pl.kernel,
        mesh=mesh,
        compiler_params=cp,
        out_type=(
            jax.ShapeDtypeStruct((B, 128), jnp.int32),
            jax.ShapeDtypeStruct((2 * B, 128), jnp.int32),
        ),
        scratch_types=[
            pltpu.VMEM((BU,), jnp.int32),
            pltpu.VMEM((BI,), jnp.int32),
            pltpu.VMEM((BU, 128), jnp.int32),
            pltpu.VMEM((BI, 128), jnp.int32),
            pltpu.SemaphoreType.DMA,
            pltpu.SemaphoreType.DMA,
        ],
    )
    def gather_kernel(
        ut_hbm, it_hbm, uid_hbm, iid_hbm, uo_hbm, io_hbm,
        uidx_v, iidx_v, urows_v, irows_v, sem_u, sem_i,
    ):
        wid = lax.axis_index("s") * NC + lax.axis_index("c")
        ubase = wid * BU
        ibase = wid * BI
        pltpu.sync_copy(uid_hbm.at[pl.ds(ubase, BU)], uidx_v)
        pltpu.sync_copy(iid_hbm.at[pl.ds(ibase, BI)], iidx_v)

        @pl.loop(0, BU // L)
        def _(j):
            s = pl.ds(j * L, L)
            uidx_v[s] = lax.shift_right_logical(uidx_v[s], 1)

        @pl.loop(0, BI // L)
        def _(j):
            s = pl.ds(j * L, L)
            iidx_v[s] = lax.shift_right_logical(iidx_v[s], 1)

        for c in range(BU // GW):
            pltpu.async_copy(
                ut_hbm.at[uidx_v.at[pl.ds(c * GW, GW)]],
                urows_v.at[pl.ds(c * GW, GW)],
                sem_u,
            )
        for c in range(BI // GW):
            pltpu.async_copy(
                it_hbm.at[iidx_v.at[pl.ds(c * GW, GW)]],
                irows_v.at[pl.ds(c * GW, GW)],
                sem_i,
            )
        pltpu.make_async_copy(
            ut_hbm.at[pl.ds(0, BU)], urows_v, sem_u
        ).wait()
        pltpu.sync_copy(urows_v, uo_hbm.at[pl.ds(ubase, BU)])
        pltpu.make_async_copy(
            it_hbm.at[pl.ds(0, BI)], irows_v, sem_i
        ).wait()
        pltpu.sync_copy(irows_v, io_hbm.at[pl.ds(ibase, BI)])

    return gather_kernel(st_user, st_item, user_id, item_ids2)


def _pick(rows_i32, ids):
    lo = pltpu.bitcast(lax.shift_left(rows_i32, 16), jnp.float32)
    hi = pltpu.bitcast(
        lax.bitwise_and(rows_i32, jnp.int32(-65536)), jnp.float32
    )
    a_bit = (ids & 1) != 0
    xa = jnp.where(a_bit, hi, lo)
    return xa[:, :D].astype(jnp.bfloat16)


def _mm_body(u_ref, p_ref, n_ref, uid_ref, pid_ref, nid_ref, pos_ref, neg_ref):
    u = _pick(u_ref[...], uid_ref[...])
    p = _pick(p_ref[...], pid_ref[...])
    n = _pick(n_ref[...], nid_ref[...])
    dims = (((1,), (1,)), ((), ()))
    pos_ref[...] = jax.lax.dot_general(
        u, p, dims, preferred_element_type=jnp.float32
    )
    neg_ref[...] = jax.lax.dot_general(
        u, n, dims, preferred_element_type=jnp.float32
    )


def kernel(user_id, item_id, neg_item, user_table, item_table, training=False):
    eye = jnp.concatenate(
        [jnp.eye(D, dtype=jnp.float32),
         jnp.zeros((D, 128 - D), jnp.float32)], axis=1,
    )
    st_user = _stage_table(user_table.T, eye)
    st_item = _stage_table(item_table.T, eye)
    item_ids2 = jnp.concatenate([item_id, neg_item])
    u_rows, i_rows = _sc_gather(st_user, st_item, user_id, item_ids2)
    uid2 = user_id.reshape(B, 1)
    pid2 = item_id.reshape(B, 1)
    nid2 = neg_item.reshape(B, 1)
    pos, neg = pl.pallas_call(
        _mm_body,
        grid=(B // BM,),
        compiler_params=pltpu.CompilerParams(
            dimension_semantics=("parallel",)
        ),
        in_specs=[
            pl.BlockSpec((BM, 128), lambda i: (i, 0)),
            pl.BlockSpec((B, 128), lambda i: (0, 0)),
            pl.BlockSpec((B, 128), lambda i: (1, 0)),
            pl.BlockSpec((BM, 1), lambda i: (i, 0)),
            pl.BlockSpec((B, 1), lambda i: (0, 0)),
            pl.BlockSpec((B, 1), lambda i: (0, 0)),
        ],
        out_specs=[
            pl.BlockSpec((BM, B), lambda i: (i, 0)),
            pl.BlockSpec((BM, B), lambda i: (i, 0)),
        ],
        out_shape=[jax.ShapeDtypeStruct((B, B), jnp.float32)] * 2,
    )(u_rows, i_rows, i_rows, uid2, pid2, nid2)
    return pos, neg


# speedup vs baseline: 1.4463x; 1.4463x over previous
import dataclasses
import functools

import jax
import jax.numpy as jnp
from jax import lax
from jax.experimental import pallas as pl
from jax.experimental.pallas import tpu as pltpu
from jax.experimental.pallas import tpu_sc as plsc

B = 4096
D = 64
NT = 1000000
NC, NS = 2, 16
NW = NC * NS
BU = B // NW
BI = 2 * B // NW
BM = 256
TC_C = 16384
N_STEPS = (NT + TC_C - 1) // TC_C
NP = N_STEPS * (TC_C // 4)
GW = 128
L = 16


def _t_body(t_ref, eye_ref, out_ref):
    a = t_ref[...]
    dims = (((0,), (0,)), ((), ()))
    h = TC_C // 2
    tl = jax.lax.dot_general(
        a[:, :h], eye_ref[...], dims, preferred_element_type=jnp.float32
    )
    tr = jax.lax.dot_general(
        a[:, h:], eye_ref[...], dims, preferred_element_type=jnp.float32
    )
    t16 = jnp.concatenate(
        [tl.astype(jnp.bfloat16), tr.astype(jnp.bfloat16)], axis=1
    )
    out_ref[...] = pltpu.bitcast(t16, jnp.int32)


def _stage_table(table_t, eye):
    return pl.pallas_call(
        _t_body,
        grid=(N_STEPS,),
        compiler_params=pltpu.CompilerParams(
            dimension_semantics=("parallel",)
        ),
        in_specs=[
            pl.BlockSpec((D, TC_C), lambda i: (0, i)),
            pl.BlockSpec((D, D), lambda i: (0, 0)),
        ],
        out_specs=pl.BlockSpec((TC_C // 4, 128), lambda i: (i, 0)),
        out_shape=jax.ShapeDtypeStruct((NP, 128), jnp.int32),
    )(table_t, eye)


def _sc_gather(st_user, st_item, user_id, item_ids2):
    mesh = plsc.VectorSubcoreMesh(core_axis_name="c", subcore_axis_name="s")
    cp = pltpu.CompilerParams()
    if "use_tc_tiling_on_sc" in pltpu.CompilerParams.__dataclass_fields__:
        cp = dataclasses.replace(cp, use_tc_tiling_on_sc=True)

    @functools.partial(
        pl.kernel,
        mesh=mesh,
        compiler_params=cp,
        out_type=(
            jax.ShapeDtypeStruct((B, 128), jnp.int32),
            jax.ShapeDtypeStruct((2 * B, 128), jnp.int32),
        ),
        scratch_types=[
            pltpu.VMEM((BU,), jnp.int32),
            pltpu.VMEM((BI,), jnp.int32),
            pltpu.VMEM((BU, 128), jnp.int32),
            pltpu.VMEM((BI, 128), jnp.int32),
            pltpu.SemaphoreType.DMA,
            pltpu.SemaphoreType.DMA,
        ],
    )
    def gather_kernel(
        ut_hbm, it_hbm, uid_hbm, iid_hbm, uo_hbm, io_hbm,
        uidx_v, iidx_v, urows_v, irows_v, sem_u, sem_i,
    ):
        wid = lax.axis_index("s") * NC + lax.axis_index("c")
        ubase = wid * BU
        ibase = wid * BI
        pltpu.sync_copy(uid_hbm.at[pl.ds(ubase, BU)], uidx_v)
        pltpu.sync_copy(iid_hbm.at[pl.ds(ibase, BI)], iidx_v)

        def to_staged(g):
            blk = lax.shift_right_logical(g, 14)
            oh = lax.bitwise_and(g, jnp.int32(TC_C // 2 - 1))
            return lax.shift_left(blk, 12) + lax.shift_right_logical(oh, 1)

        @pl.loop(0, BU // L)
        def _(j):
            s = pl.ds(j * L, L)
            uidx_v[s] = to_staged(uidx_v[s])

        @pl.loop(0, BI // L)
        def _(j):
            s = pl.ds(j * L, L)
            iidx_v[s] = to_staged(iidx_v[s])

        for c in range(BU // GW):
            pltpu.async_copy(
                ut_hbm.at[uidx_v.at[pl.ds(c * GW, GW)]],
                urows_v.at[pl.ds(c * GW, GW)],
                sem_u,
            )
        for c in range(BI // GW):
            pltpu.async_copy(
                it_hbm.at[iidx_v.at[pl.ds(c * GW, GW)]],
                irows_v.at[pl.ds(c * GW, GW)],
                sem_i,
            )
        pltpu.make_async_copy(
            ut_hbm.at[pl.ds(0, BU)], urows_v, sem_u
        ).wait()
        pltpu.sync_copy(urows_v, uo_hbm.at[pl.ds(ubase, BU)])
        pltpu.make_async_copy(
            it_hbm.at[pl.ds(0, BI)], irows_v, sem_i
        ).wait()
        pltpu.sync_copy(irows_v, io_hbm.at[pl.ds(ibase, BI)])

    return gather_kernel(st_user, st_item, user_id, item_ids2)


def _pick(rows_i32, ids):
    lo = pltpu.bitcast(lax.shift_left(rows_i32, 16), jnp.float32)
    hi = pltpu.bitcast(
        lax.bitwise_and(rows_i32, jnp.int32(-65536)), jnp.float32
    )
    a_bit = (ids & 1) != 0
    s_bit = (ids & (TC_C // 2)) != 0
    xa = jnp.where(a_bit, hi, lo)
    xs = jnp.where(s_bit, xa[:, D:], xa[:, :D])
    return xs.astype(jnp.bfloat16)


def _mm_body(u_ref, p_ref, n_ref, uid_ref, pid_ref, nid_ref, pos_ref, neg_ref):
    u = _pick(u_ref[...], uid_ref[...])
    p = _pick(p_ref[...], pid_ref[...])
    n = _pick(n_ref[...], nid_ref[...])
    dims = (((1,), (1,)), ((), ()))
    pos_ref[...] = jax.lax.dot_general(
        u, p, dims, preferred_element_type=jnp.float32
    )
    neg_ref[...] = jax.lax.dot_general(
        u, n, dims, preferred_element_type=jnp.float32
    )


def kernel(user_id, item_id, neg_item, user_table, item_table, training=False):
    eye = jnp.eye(D, dtype=jnp.float32)
    st_user = _stage_table(user_table.T, eye)
    st_item = _stage_table(item_table.T, eye)
    item_ids2 = jnp.concatenate([item_id, neg_item])
    u_rows, i_rows = _sc_gather(st_user, st_item, user_id, item_ids2)
    uid2 = user_id.reshape(B, 1)
    pid2 = item_id.reshape(B, 1)
    nid2 = neg_item.reshape(B, 1)
    pos, neg = pl.pallas_call(
        _mm_body,
        grid=(B // BM,),
        compiler_params=pltpu.CompilerParams(
            dimension_semantics=("parallel",)
        ),
        in_specs=[
            pl.BlockSpec((BM, 128), lambda i: (i, 0)),
            pl.BlockSpec((B, 128), lambda i: (0, 0)),
            pl.BlockSpec((B, 128), lambda i: (1, 0)),
            pl.BlockSpec((BM, 1), lambda i: (i, 0)),
            pl.BlockSpec((B, 1), lambda i: (0, 0)),
            pl.BlockSpec((B, 1), lambda i: (0, 0)),
        ],
        out_specs=[
            pl.BlockSpec((BM, B), lambda i: (i, 0)),
            pl.BlockSpec((BM, B), lambda i: (i, 0)),
        ],
        out_shape=[jax.ShapeDtypeStruct((B, B), jnp.float32)] * 2,
    )(u_rows, i_rows, i_rows, uid2, pid2, nid2)
    return pos, neg


# speedup vs baseline: 1.5272x; 1.0560x over previous
import dataclasses
import functools

import jax
import jax.numpy as jnp
from jax import lax
from jax.experimental import pallas as pl
from jax.experimental.pallas import tpu as pltpu
from jax.experimental.pallas import tpu_sc as plsc

B = 4096
D = 64
NT = 1000000
NC, NS = 2, 16
NW = NC * NS
BU = B // NW
BI = 2 * B // NW
BM = 256
TC_C = 16384
N_STEPS = (NT + TC_C - 1) // TC_C
NP = N_STEPS * (TC_C // 4)
GW = 128
L = 16


def _t_body(t_ref, eye_ref, out_ref):
    a = t_ref[...]
    dims = (((0,), (0,)), ((), ()))
    h = TC_C // 2
    tl = jax.lax.dot_general(
        a[:, :h], eye_ref[...], dims, preferred_element_type=jnp.float32
    )
    tr = jax.lax.dot_general(
        a[:, h:], eye_ref[...], dims, preferred_element_type=jnp.float32
    )
    t16 = jnp.concatenate(
        [tl.astype(jnp.bfloat16), tr.astype(jnp.bfloat16)], axis=1
    )
    out_ref[...] = pltpu.bitcast(t16, jnp.int32)


def _stage_table(table_t, eye):
    return pl.pallas_call(
        _t_body,
        grid=(N_STEPS,),
        compiler_params=pltpu.CompilerParams(
            dimension_semantics=("parallel",)
        ),
        in_specs=[
            pl.BlockSpec((D, TC_C), lambda i: (0, i)),
            pl.BlockSpec((D, D), lambda i: (0, 0)),
        ],
        out_specs=pl.BlockSpec((TC_C // 4, 128), lambda i: (i, 0)),
        out_shape=jax.ShapeDtypeStruct((NP, 128), jnp.int32),
    )(table_t, eye)


def _sc_gather(st_user, st_item, user_id, item_ids2):
    mesh = plsc.VectorSubcoreMesh(core_axis_name="c", subcore_axis_name="s")
    cp = pltpu.CompilerParams()
    if "use_tc_tiling_on_sc" in pltpu.CompilerParams.__dataclass_fields__:
        cp = dataclasses.replace(cp, use_tc_tiling_on_sc=True)

    @functools.partial(
        pl.kernel,
        mesh=mesh,
        compiler_params=cp,
        out_type=(
            jax.ShapeDtypeStruct((B, 128), jnp.int32),
            jax.ShapeDtypeStruct((2 * B, 128), jnp.int32),
        ),
        scratch_types=[
            pltpu.VMEM((BU,), jnp.int32),
            pltpu.VMEM((BI,), jnp.int32),
            pltpu.VMEM((BU, 128), jnp.int32),
            pltpu.VMEM((BI, 128), jnp.int32),
            pltpu.SemaphoreType.DMA,
            pltpu.SemaphoreType.DMA,
        ],
    )
    def gather_kernel(
        ut_hbm, it_hbm, uid_hbm, iid_hbm, uo_hbm, io_hbm,
        uidx_v, iidx_v, urows_v, irows_v, sem_u, sem_i,
    ):
        wid = lax.axis_index("s") * NC + lax.axis_index("c")
        ubase = wid * BU
        ibase = wid * BI
        pltpu.sync_copy(uid_hbm.at[pl.ds(ubase, BU)], uidx_v)
        pltpu.sync_copy(iid_hbm.at[pl.ds(ibase, BI)], iidx_v)

        def to_staged(g):
            blk = lax.shift_right_logical(g, 14)
            oh = lax.bitwise_and(g, jnp.int32(TC_C // 2 - 1))
            return lax.shift_left(blk, 12) + lax.shift_right_logical(oh, 1)

        @pl.loop(0, BU // L)
        def _(j):
            s = pl.ds(j * L, L)
            uidx_v[s] = to_staged(uidx_v[s])

        @pl.loop(0, BI // L)
        def _(j):
            s = pl.ds(j * L, L)
            iidx_v[s] = to_staged(iidx_v[s])

        for c in range(BU // GW):
            pltpu.async_copy(
                ut_hbm.at[uidx_v.at[pl.ds(c * GW, GW)]],
                urows_v.at[pl.ds(c * GW, GW)],
                sem_u,
            )
        for c in range(BI // GW):
            pltpu.async_copy(
                it_hbm.at[iidx_v.at[pl.ds(c * GW, GW)]],
                irows_v.at[pl.ds(c * GW, GW)],
                sem_i,
            )
        pltpu.make_async_copy(
            ut_hbm.at[pl.ds(0, BU)], urows_v, sem_u
        ).wait()
        pltpu.sync_copy(urows_v, uo_hbm.at[pl.ds(ubase, BU)])
        pltpu.make_async_copy(
            it_hbm.at[pl.ds(0, BI)], irows_v, sem_i
        ).wait()
        pltpu.sync_copy(irows_v, io_hbm.at[pl.ds(ibase, BI)])

    return gather_kernel(st_user, st_item, user_id, item_ids2)


def _pick(rows_i32, ids):
    lo = pltpu.bitcast(lax.shift_left(rows_i32, 16), jnp.float32)
    hi = pltpu.bitcast(
        lax.bitwise_and(rows_i32, jnp.int32(-65536)), jnp.float32
    )
    a_bit = (ids & 1) != 0
    s_bit = (ids & (TC_C // 2)) != 0
    xa = jnp.where(a_bit, hi, lo)
    xs = jnp.where(s_bit, xa[:, D:], xa[:, :D])
    return xs.astype(jnp.bfloat16)


def _mm_body(u_ref, p_ref, n_ref, uid_ref, pid_ref, nid_ref, pos_ref, neg_ref,
             p_sc, n_sc):
    @pl.when(pl.program_id(0) == 0)
    def _():
        p_sc[...] = _pick(p_ref[...], pid_ref[...])
        n_sc[...] = _pick(n_ref[...], nid_ref[...])

    u = _pick(u_ref[...], uid_ref[...])
    dims = (((1,), (1,)), ((), ()))
    pos_ref[...] = jax.lax.dot_general(
        u, p_sc[...], dims, preferred_element_type=jnp.float32
    )
    neg_ref[...] = jax.lax.dot_general(
        u, n_sc[...], dims, preferred_element_type=jnp.float32
    )


def kernel(user_id, item_id, neg_item, user_table, item_table, training=False):
    eye = jnp.eye(D, dtype=jnp.float32)
    st_user = _stage_table(user_table.T, eye)
    st_item = _stage_table(item_table.T, eye)
    item_ids2 = jnp.concatenate([item_id, neg_item])
    u_rows, i_rows = _sc_gather(st_user, st_item, user_id, item_ids2)
    uid2 = user_id.reshape(B, 1)
    pid2 = item_id.reshape(B, 1)
    nid2 = neg_item.reshape(B, 1)
    pos, neg = pl.pallas_call(
        _mm_body,
        grid=(B // BM,),
        compiler_params=pltpu.CompilerParams(
            dimension_semantics=("parallel",)
        ),
        in_specs=[
            pl.BlockSpec((BM, 128), lambda i: (i, 0)),
            pl.BlockSpec((B, 128), lambda i: (0, 0)),
            pl.BlockSpec((B, 128), lambda i: (1, 0)),
            pl.BlockSpec((BM, 1), lambda i: (i, 0)),
            pl.BlockSpec((B, 1), lambda i: (0, 0)),
            pl.BlockSpec((B, 1), lambda i: (0, 0)),
        ],
        out_specs=[
            pl.BlockSpec((BM, B), lambda i: (i, 0)),
            pl.BlockSpec((BM, B), lambda i: (i, 0)),
        ],
        out_shape=[jax.ShapeDtypeStruct((B, B), jnp.float32)] * 2,
        scratch_shapes=[
            pltpu.VMEM((B, D), jnp.bfloat16),
            pltpu.VMEM((B, D), jnp.bfloat16),
        ],
    )(u_rows, i_rows, i_rows, uid2, pid2, nid2)
    return pos, neg
